# initial kernel scaffold (unmeasured)
import jax
import jax.numpy as jnp
from jax import lax
from jax.experimental import pallas as pl
from jax.experimental.pallas import tpu as pltpu

N_DEV = 4
BM = 1024
EPS = 1e-5


def kernel(x, gamma):
    m, n_local = x.shape
    nb = m // BM
    g2 = gamma.reshape(1, n_local)

    def body(x_hbm, g_ref, o_hbm, xv, yv, total, comm,
             in_sems, out_sems, send_sems, recv_sems):
        my = lax.axis_index("i")

        def load(b, slot):
            return pltpu.make_async_copy(
                x_hbm.at[pl.ds(b * BM, BM), :], xv.at[slot],
                in_sems.at[slot])

        def store(b, slot):
            return pltpu.make_async_copy(
                yv.at[slot], o_hbm.at[pl.ds(b * BM, BM), :],
                out_sems.at[slot])

        ones_row = jnp.ones((1, n_local), jnp.float32)

        load(0, 0).start()
        for b in range(nb):
            if b + 1 < nb:
                load(b + 1, (b + 1) % 2).start()
            load(b, b % 2).wait()
            xb = xv[b % 2]
            prow = lax.dot_general(
                ones_row, xb * xb, (((1,), (1,)), ((), ())),
                preferred_element_type=jnp.float32,
                precision=lax.Precision.HIGHEST)
            total[pl.ds(b, 1), :] = prow

        load(0, 0).start()
        load(1, 1).start()

        barrier = pltpu.get_barrier_semaphore()
        for k in range(1, N_DEV):
            pl.semaphore_signal(
                barrier, inc=1, device_id=((my + k) % N_DEV,),
                device_id_type=pl.DeviceIdType.MESH)
        pl.semaphore_wait(barrier, N_DEV - 1)

        rdmas = []
        for k in range(1, N_DEV):
            slot = N_DEV - 1 - k
            rdma = pltpu.make_async_remote_copy(
                src_ref=total,
                dst_ref=comm.at[slot],
                send_sem=send_sems.at[k - 1],
                recv_sem=recv_sems.at[slot],
                device_id=((my + k) % N_DEV,),
                device_id_type=pl.DeviceIdType.MESH)
            rdma.start()
            rdmas.append(rdma)
        for rdma in rdmas:
            rdma.wait_recv()
        for rdma in rdmas:
            rdma.wait_send()

        t = total[:, :] + comm[0] + comm[1] + comm[2]
        inv = lax.rsqrt(
            t * jnp.float32(1.0 / (N_DEV * n_local)) + jnp.float32(EPS))
        ident = (lax.broadcasted_iota(jnp.int32, (128, 128), 0) ==
                 lax.broadcasted_iota(jnp.int32, (128, 128), 1)
                 ).astype(jnp.float32)
        cols = [
            lax.dot_general(
                ident, inv[:, k * 128:(k + 1) * 128],
                (((1,), (1,)), ((), ())),
                preferred_element_type=jnp.float32,
                precision=lax.Precision.HIGHEST)
            for k in range(BM // 128)
        ]
        scale_cols = jnp.concatenate(cols, axis=0)

        g = g_ref[:, :]
        for b in range(nb):
            slot = b % 2
            if b >= 2:
                store(b - 2, slot).wait()
            load(b, slot).wait()
            col = scale_cols[:, b:b + 1]
            yv[slot, :, :] = xv[slot] * col * g
            store(b, slot).start()
            if b + 2 < nb:
                load(b + 2, slot).start()
        store(nb - 2, (nb - 2) % 2).wait()
        store(nb - 1, (nb - 1) % 2).wait()

    return pl.pallas_call(
        body,
        out_shape=jax.ShapeDtypeStruct((m, n_local), jnp.float32),
        in_specs=[pl.BlockSpec(memory_space=pltpu.ANY),
                  pl.BlockSpec(memory_space=pltpu.VMEM)],
        out_specs=pl.BlockSpec(memory_space=pltpu.ANY),
        scratch_shapes=[
            pltpu.VMEM((2, BM, n_local), jnp.float32),
            pltpu.VMEM((2, BM, n_local), jnp.float32),
            pltpu.VMEM((8, BM), jnp.float32),
            pltpu.VMEM((N_DEV - 1, 8, BM), jnp.float32),
            pltpu.SemaphoreType.DMA((2,)),
            pltpu.SemaphoreType.DMA((2,)),
            pltpu.SemaphoreType.DMA((N_DEV - 1,)),
            pltpu.SemaphoreType.DMA((N_DEV - 1,)),
        ],
        compiler_params=pltpu.CompilerParams(collective_id=0),
    )(x, g2)


# baseline (device time: 148448 ns/iter reference)
import jax
import jax.numpy as jnp
from jax import lax
from jax.experimental import pallas as pl
from jax.experimental.pallas import tpu as pltpu

N_DEV = 4
BM = 512
EPS = 1e-5


def kernel(x, gamma):
    m, n_local = x.shape
    nb = m // BM
    g2 = gamma.reshape(1, n_local)

    def body(x_hbm, g_ref, o_hbm, xv, yv, total, comm,
             in_sems, out_sems, send_sems, recv_sems):
        my = lax.axis_index("i")

        def load(b, slot):
            return pltpu.make_async_copy(
                x_hbm.at[pl.ds(b * BM, BM), :], xv.at[slot],
                in_sems.at[slot])

        def store(b, slot):
            return pltpu.make_async_copy(
                yv.at[slot], o_hbm.at[pl.ds(b * BM, BM), :],
                out_sems.at[slot])

        ones_row = jnp.ones((1, n_local), jnp.float32)

        load(0, 0).start()
        for b in range(nb):
            if b + 1 < nb:
                load(b + 1, (b + 1) % 2).start()
            load(b, b % 2).wait()
            xb = xv[b % 2]
            prow = lax.dot_general(
                ones_row, xb * xb, (((1,), (1,)), ((), ())),
                preferred_element_type=jnp.float32,
                precision=lax.Precision.HIGHEST)
            total[pl.ds(b, 1), :] = prow

        load(0, 0).start()
        load(1, 1).start()

        barrier = pltpu.get_barrier_semaphore()
        for k in range(1, N_DEV):
            pl.semaphore_signal(
                barrier, inc=1, device_id=((my + k) % N_DEV,),
                device_id_type=pl.DeviceIdType.MESH)
        pl.semaphore_wait(barrier, N_DEV - 1)

        rdmas = []
        for k in range(1, N_DEV):
            slot = N_DEV - 1 - k
            rdma = pltpu.make_async_remote_copy(
                src_ref=total,
                dst_ref=comm.at[slot],
                send_sem=send_sems.at[k - 1],
                recv_sem=recv_sems.at[slot],
                device_id=((my + k) % N_DEV,),
                device_id_type=pl.DeviceIdType.MESH)
            rdma.start()
            rdmas.append(rdma)
        for rdma in rdmas:
            rdma.wait_recv()
        for rdma in rdmas:
            rdma.wait_send()

        t = total[:, :] + comm[0] + comm[1] + comm[2]
        inv = lax.rsqrt(
            t * jnp.float32(1.0 / (N_DEV * n_local)) + jnp.float32(EPS))
        ident = (lax.broadcasted_iota(jnp.int32, (128, 128), 0) ==
                 lax.broadcasted_iota(jnp.int32, (128, 128), 1)
                 ).astype(jnp.float32)
        cols = [
            lax.dot_general(
                ident, inv[:, k * 128:(k + 1) * 128],
                (((1,), (1,)), ((), ())),
                preferred_element_type=jnp.float32,
                precision=lax.Precision.HIGHEST)
            for k in range(BM // 128)
        ]
        scale_cols = jnp.concatenate(cols, axis=0)

        g = g_ref[:, :]
        for b in range(nb):
            slot = b % 2
            if b >= 2:
                store(b - 2, slot).wait()
            load(b, slot).wait()
            col = scale_cols[:, b:b + 1]
            yv[slot, :, :] = xv[slot] * col * g
            store(b, slot).start()
            if b + 2 < nb:
                load(b + 2, slot).start()
        store(nb - 2, (nb - 2) % 2).wait()
        store(nb - 1, (nb - 1) % 2).wait()

    return pl.pallas_call(
        body,
        out_shape=jax.ShapeDtypeStruct((m, n_local), jnp.float32),
        in_specs=[pl.BlockSpec(memory_space=pl.ANY),
                  pl.BlockSpec(memory_space=pltpu.VMEM)],
        out_specs=pl.BlockSpec(memory_space=pl.ANY),
        scratch_shapes=[
            pltpu.VMEM((2, BM, n_local), jnp.float32),
            pltpu.VMEM((2, BM, n_local), jnp.float32),
            pltpu.VMEM((nb, BM), jnp.float32),
            pltpu.VMEM((N_DEV - 1, nb, BM), jnp.float32),
            pltpu.SemaphoreType.DMA((2,)),
            pltpu.SemaphoreType.DMA((2,)),
            pltpu.SemaphoreType.DMA((N_DEV - 1,)),
            pltpu.SemaphoreType.DMA((N_DEV - 1,)),
        ],
        compiler_params=pltpu.CompilerParams(collective_id=0),
    )(x, g2)


# device time: 72971 ns/iter; 2.0343x vs baseline; 2.0343x over previous
import jax
import jax.numpy as jnp
from jax import lax
from jax.experimental import pallas as pl
from jax.experimental.pallas import tpu as pltpu

N_DEV = 4
BM = 512
EPS = 1e-5


def _partials_allreduce(x):
    m, n_local = x.shape
    nb = m // BM

    def body(x_hbm, tot_out, xv, total, comm, in_sems, send_sems, recv_sems):
        my = lax.axis_index("i")

        def load(b, slot):
            return pltpu.make_async_copy(
                x_hbm.at[pl.ds(b * BM, BM), :], xv.at[slot],
                in_sems.at[slot])

        ones_row = jnp.ones((1, n_local), jnp.bfloat16)

        load(0, 0).start()
        for b in range(nb):
            if b + 1 < nb:
                load(b + 1, (b + 1) % 2).start()
            load(b, b % 2).wait()
            xb = xv[b % 2]
            xsq = (xb * xb).astype(jnp.bfloat16)
            prow = lax.dot_general(
                ones_row, xsq, (((1,), (1,)), ((), ())),
                preferred_element_type=jnp.float32)
            total[pl.ds(b, 1), :] = prow

        barrier = pltpu.get_barrier_semaphore()
        for k in range(1, N_DEV):
            pl.semaphore_signal(
                barrier, inc=1, device_id=((my + k) % N_DEV,),
                device_id_type=pl.DeviceIdType.MESH)
        pl.semaphore_wait(barrier, N_DEV - 1)

        rdmas = []
        for k in range(1, N_DEV):
            slot = N_DEV - 1 - k
            rdma = pltpu.make_async_remote_copy(
                src_ref=total,
                dst_ref=comm.at[slot],
                send_sem=send_sems.at[k - 1],
                recv_sem=recv_sems.at[slot],
                device_id=((my + k) % N_DEV,),
                device_id_type=pl.DeviceIdType.MESH)
            rdma.start()
            rdmas.append(rdma)
        for rdma in rdmas:
            rdma.wait_recv()
        for rdma in rdmas:
            rdma.wait_send()

        tot_out[:, :] = total[:, :] + comm[0] + comm[1] + comm[2]

    return pl.pallas_call(
        body,
        out_shape=jax.ShapeDtypeStruct((nb, BM), jnp.float32),
        in_specs=[pl.BlockSpec(memory_space=pl.ANY)],
        out_specs=pl.BlockSpec(memory_space=pltpu.VMEM),
        scratch_shapes=[
            pltpu.VMEM((2, BM, n_local), jnp.float32),
            pltpu.VMEM((nb, BM), jnp.float32),
            pltpu.VMEM((N_DEV - 1, nb, BM), jnp.float32),
            pltpu.SemaphoreType.DMA((2,)),
            pltpu.SemaphoreType.DMA((N_DEV - 1,)),
            pltpu.SemaphoreType.DMA((N_DEV - 1,)),
        ],
        compiler_params=pltpu.CompilerParams(collective_id=0),
    )(x)


def _normalize(x, total, g2):
    m, n_local = x.shape
    nb = m // BM

    def body(x_hbm, tot_ref, g_ref, o_hbm, xv, yv, in_sems, out_sems):
        def load(b, slot):
            return pltpu.make_async_copy(
                x_hbm.at[pl.ds(b * BM, BM), :], xv.at[slot],
                in_sems.at[slot])

        def store(b, slot):
            return pltpu.make_async_copy(
                yv.at[slot], o_hbm.at[pl.ds(b * BM, BM), :],
                out_sems.at[slot])

        load(0, 0).start()
        load(1, 1).start()

        inv = lax.rsqrt(
            tot_ref[:, :] * jnp.float32(1.0 / (N_DEV * n_local))
            + jnp.float32(EPS))
        ident = (lax.broadcasted_iota(jnp.int32, (128, 128), 0) ==
                 lax.broadcasted_iota(jnp.int32, (128, 128), 1)
                 ).astype(jnp.float32)
        cols = [
            lax.dot_general(
                ident, inv[:, k * 128:(k + 1) * 128],
                (((1,), (1,)), ((), ())),
                preferred_element_type=jnp.float32,
                precision=lax.Precision.HIGHEST)
            for k in range(BM // 128)
        ]
        scale_cols = jnp.concatenate(cols, axis=0)

        g = g_ref[:, :]
        for b in range(nb):
            slot = b % 2
            if b >= 2:
                store(b - 2, slot).wait()
            load(b, slot).wait()
            col = scale_cols[:, b:b + 1]
            yv[slot, :, :] = xv[slot] * col * g
            store(b, slot).start()
            if b + 2 < nb:
                load(b + 2, slot).start()
        store(nb - 2, (nb - 2) % 2).wait()
        store(nb - 1, (nb - 1) % 2).wait()

    return pl.pallas_call(
        body,
        out_shape=jax.ShapeDtypeStruct((m, n_local), jnp.float32),
        in_specs=[pl.BlockSpec(memory_space=pl.ANY),
                  pl.BlockSpec(memory_space=pltpu.VMEM),
                  pl.BlockSpec(memory_space=pltpu.VMEM)],
        out_specs=pl.BlockSpec(memory_space=pl.ANY),
        scratch_shapes=[
            pltpu.VMEM((2, BM, n_local), jnp.float32),
            pltpu.VMEM((2, BM, n_local), jnp.float32),
            pltpu.SemaphoreType.DMA((2,)),
            pltpu.SemaphoreType.DMA((2,)),
        ],
    )(x, total, g2)


def kernel(x, gamma):
    m, n_local = x.shape
    total = _partials_allreduce(x)
    return _normalize(x, total, gamma.reshape(1, n_local))


# device time: 70795 ns/iter; 2.0969x vs baseline; 1.0307x over previous
import jax
import jax.numpy as jnp
from jax import lax
from jax.experimental import pallas as pl
from jax.experimental.pallas import tpu as pltpu

N_DEV = 4
BM = 512
EPS = 1e-5


def _partials_allreduce(x):
    m, n_local = x.shape
    nb = m // BM

    def body(x_hbm, tot_out, xv, total, comm, in_sems, send_sems, recv_sems):
        my = lax.axis_index("i")

        def load(b, slot):
            return pltpu.make_async_copy(
                x_hbm.at[pl.ds(b * BM, BM), :], xv.at[slot],
                in_sems.at[slot])

        def half_rdmas(h):
            rows = pl.ds(h * (nb // 2), nb // 2)
            out = []
            for k in range(1, N_DEV):
                slot = N_DEV - 1 - k
                out.append(pltpu.make_async_remote_copy(
                    src_ref=total.at[rows, :],
                    dst_ref=comm.at[slot, rows, :],
                    send_sem=send_sems.at[h, k - 1],
                    recv_sem=recv_sems.at[h, slot],
                    device_id=((my + k) % N_DEV,),
                    device_id_type=pl.DeviceIdType.MESH))
            return out

        barrier = pltpu.get_barrier_semaphore()
        for k in range(1, N_DEV):
            pl.semaphore_signal(
                barrier, inc=1, device_id=((my + k) % N_DEV,),
                device_id_type=pl.DeviceIdType.MESH)

        ones_row = jnp.ones((1, n_local), jnp.bfloat16)

        rdmas = []
        for b in range(min(4, nb)):
            load(b, b).start()
        for b in range(nb):
            load(b, b % 4).wait()
            if b + 4 < nb:
                load(b + 4, b % 4).start()
            xb = xv[b % 4]
            xsq = (xb * xb).astype(jnp.bfloat16)
            prow = lax.dot_general(
                ones_row, xsq, (((1,), (1,)), ((), ())),
                preferred_element_type=jnp.float32)
            total[pl.ds(b, 1), :] = prow
            if b == nb // 2 - 1:
                pl.semaphore_wait(barrier, N_DEV - 1)
                rdmas = half_rdmas(0)
                for r in rdmas:
                    r.start()
        for r in half_rdmas(1):
            r.start()
            rdmas.append(r)
        for rdma in rdmas:
            rdma.wait_recv()
        for rdma in rdmas:
            rdma.wait_send()

        tot_out[:, :] = total[:, :] + comm[0] + comm[1] + comm[2]

    return pl.pallas_call(
        body,
        out_shape=jax.ShapeDtypeStruct((nb, BM), jnp.float32),
        in_specs=[pl.BlockSpec(memory_space=pl.ANY)],
        out_specs=pl.BlockSpec(memory_space=pltpu.VMEM),
        scratch_shapes=[
            pltpu.VMEM((4, BM, n_local), jnp.float32),
            pltpu.VMEM((nb, BM), jnp.float32),
            pltpu.VMEM((N_DEV - 1, nb, BM), jnp.float32),
            pltpu.SemaphoreType.DMA((4,)),
            pltpu.SemaphoreType.DMA((2, N_DEV - 1)),
            pltpu.SemaphoreType.DMA((2, N_DEV - 1)),
        ],
        compiler_params=pltpu.CompilerParams(collective_id=0),
    )(x)


def _normalize(x, total, g2):
    m, n_local = x.shape
    nb = m // BM

    def body(x_hbm, tot_ref, g_ref, o_hbm, xv, yv, in_sems, out_sems):
        def load(b, slot):
            return pltpu.make_async_copy(
                x_hbm.at[pl.ds(b * BM, BM), :], xv.at[slot],
                in_sems.at[slot])

        def store(b, slot):
            return pltpu.make_async_copy(
                yv.at[slot], o_hbm.at[pl.ds(b * BM, BM), :],
                out_sems.at[slot])

        load(0, 0).start()
        load(1, 1).start()

        inv = lax.rsqrt(
            tot_ref[:, :] * jnp.float32(1.0 / (N_DEV * n_local))
            + jnp.float32(EPS))
        ident = (lax.broadcasted_iota(jnp.int32, (128, 128), 0) ==
                 lax.broadcasted_iota(jnp.int32, (128, 128), 1)
                 ).astype(jnp.float32)
        cols = [
            lax.dot_general(
                ident, inv[:, k * 128:(k + 1) * 128],
                (((1,), (1,)), ((), ())),
                preferred_element_type=jnp.float32,
                precision=lax.Precision.HIGHEST)
            for k in range(BM // 128)
        ]
        scale_cols = jnp.concatenate(cols, axis=0)

        g = g_ref[:, :]
        for b in range(nb):
            slot = b % 2
            if b >= 2:
                store(b - 2, slot).wait()
            load(b, slot).wait()
            col = scale_cols[:, b:b + 1]
            yv[slot, :, :] = xv[slot] * col * g
            store(b, slot).start()
            if b + 2 < nb:
                load(b + 2, slot).start()
        store(nb - 2, (nb - 2) % 2).wait()
        store(nb - 1, (nb - 1) % 2).wait()

    return pl.pallas_call(
        body,
        out_shape=jax.ShapeDtypeStruct((m, n_local), jnp.float32),
        in_specs=[pl.BlockSpec(memory_space=pl.ANY),
                  pl.BlockSpec(memory_space=pltpu.VMEM),
                  pl.BlockSpec(memory_space=pltpu.VMEM)],
        out_specs=pl.BlockSpec(memory_space=pl.ANY),
        scratch_shapes=[
            pltpu.VMEM((2, BM, n_local), jnp.float32),
            pltpu.VMEM((2, BM, n_local), jnp.float32),
            pltpu.SemaphoreType.DMA((2,)),
            pltpu.SemaphoreType.DMA((2,)),
        ],
    )(x, total, g2)


def kernel(x, gamma):
    m, n_local = x.shape
    total = _partials_allreduce(x)
    return _normalize(x, total, gamma.reshape(1, n_local))


# device time: 70764 ns/iter; 2.0978x vs baseline; 1.0004x over previous
import jax
import jax.numpy as jnp
from jax import lax
from jax.experimental import pallas as pl
from jax.experimental.pallas import tpu as pltpu

N_DEV = 4
BM = 512
EPS = 1e-5


def _partials_allreduce(x):
    m, n_local = x.shape
    nb = m // BM

    def body(x_hbm, tot_out, xv, total, comm, in_sems, send_sems, recv_sems):
        my = lax.axis_index("i")

        def load(b, slot):
            return pltpu.make_async_copy(
                x_hbm.at[pl.ds(b * BM, BM), :], xv.at[slot],
                in_sems.at[slot])

        def half_rdmas(h):
            rows = pl.ds(h * (nb // 2), nb // 2)
            out = []
            for k in range(1, N_DEV):
                slot = N_DEV - 1 - k
                out.append(pltpu.make_async_remote_copy(
                    src_ref=total.at[rows, :],
                    dst_ref=comm.at[slot, rows, :],
                    send_sem=send_sems.at[h, k - 1],
                    recv_sem=recv_sems.at[h, slot],
                    device_id=((my + k) % N_DEV,),
                    device_id_type=pl.DeviceIdType.MESH))
            return out

        barrier = pltpu.get_barrier_semaphore()
        for k in range(1, N_DEV):
            pl.semaphore_signal(
                barrier, inc=1, device_id=((my + k) % N_DEV,),
                device_id_type=pl.DeviceIdType.MESH)

        ones_row = jnp.ones((1, n_local), jnp.bfloat16)

        rdmas = []
        for b in range(min(4, nb)):
            load(b, b).start()
        for b in range(nb):
            load(b, b % 4).wait()
            if b + 4 < nb:
                load(b + 4, b % 4).start()
            xb = xv[b % 4]
            xsq = (xb * xb).astype(jnp.bfloat16)
            prow = lax.dot_general(
                ones_row, xsq, (((1,), (1,)), ((), ())),
                preferred_element_type=jnp.float32)
            total[pl.ds(b, 1), :] = prow
            if b == nb // 2 - 1:
                pl.semaphore_wait(barrier, N_DEV - 1)
                rdmas = half_rdmas(0)
                for r in rdmas:
                    r.start()
        for r in half_rdmas(1):
            r.start()
            rdmas.append(r)
        for rdma in rdmas:
            rdma.wait_recv()
        for rdma in rdmas:
            rdma.wait_send()

        tot_out[:, :] = total[:, :] + comm[0] + comm[1] + comm[2]

    return pl.pallas_call(
        body,
        out_shape=jax.ShapeDtypeStruct((nb, BM), jnp.float32),
        in_specs=[pl.BlockSpec(memory_space=pl.ANY)],
        out_specs=pl.BlockSpec(memory_space=pltpu.VMEM),
        scratch_shapes=[
            pltpu.VMEM((4, BM, n_local), jnp.float32),
            pltpu.VMEM((nb, BM), jnp.float32),
            pltpu.VMEM((N_DEV - 1, nb, BM), jnp.float32),
            pltpu.SemaphoreType.DMA((4,)),
            pltpu.SemaphoreType.DMA((2, N_DEV - 1)),
            pltpu.SemaphoreType.DMA((2, N_DEV - 1)),
        ],
        compiler_params=pltpu.CompilerParams(collective_id=0),
    )(x)


def _normalize(x, total, g2):
    m, n_local = x.shape
    bmb = 2 * BM
    nbb = m // bmb
    nb = m // BM

    def body(x_hbm, tot_ref, g_ref, o_hbm, xv, yv, in_sems, out_sems):
        def load(b, slot):
            return pltpu.make_async_copy(
                x_hbm.at[pl.ds(b * bmb, bmb), :], xv.at[slot],
                in_sems.at[slot])

        def store(b, slot):
            return pltpu.make_async_copy(
                yv.at[slot], o_hbm.at[pl.ds(b * bmb, bmb), :],
                out_sems.at[slot])

        load(0, 0).start()
        load(1, 1).start()

        inv = lax.rsqrt(
            tot_ref[:, :] * jnp.float32(1.0 / (N_DEV * n_local))
            + jnp.float32(EPS))
        ident = (lax.broadcasted_iota(jnp.int32, (128, 128), 0) ==
                 lax.broadcasted_iota(jnp.int32, (128, 128), 1)
                 ).astype(jnp.float32)
        cols = [
            lax.dot_general(
                ident, inv[:, k * 128:(k + 1) * 128],
                (((1,), (1,)), ((), ())),
                preferred_element_type=jnp.float32,
                precision=lax.Precision.HIGHEST)
            for k in range(BM // 128)
        ]
        scale_cols = jnp.concatenate(cols, axis=0)

        g = g_ref[:, :]
        for b in range(nbb):
            slot = b % 2
            if b >= 2:
                store(b - 2, slot).wait()
            load(b, slot).wait()
            col = jnp.concatenate(
                [scale_cols[:, 2 * b:2 * b + 1],
                 scale_cols[:, 2 * b + 1:2 * b + 2]], axis=0)
            yv[slot, :, :] = xv[slot] * col * g
            store(b, slot).start()
            if b + 2 < nbb:
                load(b + 2, slot).start()
        store(nbb - 2, (nbb - 2) % 2).wait()
        store(nbb - 1, (nbb - 1) % 2).wait()

    return pl.pallas_call(
        body,
        out_shape=jax.ShapeDtypeStruct((m, n_local), jnp.float32),
        in_specs=[pl.BlockSpec(memory_space=pl.ANY),
                  pl.BlockSpec(memory_space=pltpu.VMEM),
                  pl.BlockSpec(memory_space=pltpu.VMEM)],
        out_specs=pl.BlockSpec(memory_space=pl.ANY),
        scratch_shapes=[
            pltpu.VMEM((2, bmb, n_local), jnp.float32),
            pltpu.VMEM((2, bmb, n_local), jnp.float32),
            pltpu.SemaphoreType.DMA((2,)),
            pltpu.SemaphoreType.DMA((2,)),
        ],
        compiler_params=pltpu.CompilerParams(
            vmem_limit_bytes=100 * 1024 * 1024),
    )(x, total, g2)


def kernel(x, gamma):
    m, n_local = x.shape
    total = _partials_allreduce(x)
    return _normalize(x, total, gamma.reshape(1, n_local))


# device time: 70673 ns/iter; 2.1005x vs baseline; 1.0013x over previous
import jax
import jax.numpy as jnp
from jax import lax
from jax.experimental import pallas as pl
from jax.experimental.pallas import tpu as pltpu

N_DEV = 4
BM = 512
EPS = 1e-5


def _partials_allreduce(x):
    m, n_local = x.shape
    nb = m // BM

    def body(x_hbm, tot_out, xv, total, comm, in_sems, send_sems, recv_sems):
        my = lax.axis_index("i")

        def load(b, slot):
            return pltpu.make_async_copy(
                x_hbm.at[pl.ds(b * BM, BM), :], xv.at[slot],
                in_sems.at[slot])

        def half_rdmas(h):
            rows = pl.ds(h * (nb // 2), nb // 2)
            out = []
            for k in range(1, N_DEV):
                slot = N_DEV - 1 - k
                out.append(pltpu.make_async_remote_copy(
                    src_ref=total.at[rows, :],
                    dst_ref=comm.at[slot, rows, :],
                    send_sem=send_sems.at[h, k - 1],
                    recv_sem=recv_sems.at[h, slot],
                    device_id=((my + k) % N_DEV,),
                    device_id_type=pl.DeviceIdType.MESH))
            return out

        barrier = pltpu.get_barrier_semaphore()
        for k in range(1, N_DEV):
            pl.semaphore_signal(
                barrier, inc=1, device_id=((my + k) % N_DEV,),
                device_id_type=pl.DeviceIdType.MESH)

        ones_row = jnp.ones((1, n_local), jnp.bfloat16)

        rdmas = []
        for b in range(min(4, nb)):
            load(b, b).start()
        for b in range(nb):
            load(b, b % 4).wait()
            xb = xv[b % 4]
            xsq = (xb * xb).astype(jnp.bfloat16)
            prow = lax.dot_general(
                ones_row, xsq, (((1,), (1,)), ((), ())),
                preferred_element_type=jnp.float32)
            total[pl.ds(b, 1), :] = prow
            if b + 4 < nb:
                load(b + 4, b % 4).start()
            if b == nb // 2 - 1:
                pl.semaphore_wait(barrier, N_DEV - 1)
                rdmas = half_rdmas(0)
                for r in rdmas:
                    r.start()
        for r in half_rdmas(1):
            r.start()
            rdmas.append(r)
        for rdma in rdmas:
            rdma.wait_recv()
        for rdma in rdmas:
            rdma.wait_send()

        tot_out[:, :] = total[:, :] + comm[0] + comm[1] + comm[2]

    return pl.pallas_call(
        body,
        out_shape=jax.ShapeDtypeStruct((nb, BM), jnp.float32),
        in_specs=[pl.BlockSpec(memory_space=pl.ANY)],
        out_specs=pl.BlockSpec(memory_space=pltpu.VMEM),
        scratch_shapes=[
            pltpu.VMEM((4, BM, n_local), jnp.float32),
            pltpu.VMEM((nb, BM), jnp.float32),
            pltpu.VMEM((N_DEV - 1, nb, BM), jnp.float32),
            pltpu.SemaphoreType.DMA((4,)),
            pltpu.SemaphoreType.DMA((2, N_DEV - 1)),
            pltpu.SemaphoreType.DMA((2, N_DEV - 1)),
        ],
        compiler_params=pltpu.CompilerParams(collective_id=0),
    )(x)


def _normalize(x, total, g2):
    m, n_local = x.shape
    bmb = 2 * BM
    nbb = m // bmb
    nb = m // BM

    def body(x_hbm, tot_ref, g_ref, o_hbm, xv, yv, in_sems, out_sems):
        def load(b, slot):
            return pltpu.make_async_copy(
                x_hbm.at[pl.ds(b * bmb, bmb), :], xv.at[slot],
                in_sems.at[slot])

        def store(b, slot):
            return pltpu.make_async_copy(
                yv.at[slot], o_hbm.at[pl.ds(b * bmb, bmb), :],
                out_sems.at[slot])

        load(0, 0).start()
        load(1, 1).start()

        inv = lax.rsqrt(
            tot_ref[:, :] * jnp.float32(1.0 / (N_DEV * n_local))
            + jnp.float32(EPS))
        ident = (lax.broadcasted_iota(jnp.int32, (128, 128), 0) ==
                 lax.broadcasted_iota(jnp.int32, (128, 128), 1)
                 ).astype(jnp.float32)
        cols = [
            lax.dot_general(
                ident, inv[:, k * 128:(k + 1) * 128],
                (((1,), (1,)), ((), ())),
                preferred_element_type=jnp.float32,
                precision=lax.Precision.HIGHEST)
            for k in range(BM // 128)
        ]
        scale_cols = jnp.concatenate(cols, axis=0)

        g = g_ref[:, :]
        for b in range(nbb):
            slot = b % 2
            if b >= 2:
                store(b - 2, slot).wait()
            load(b, slot).wait()
            col = jnp.concatenate(
                [scale_cols[:, 2 * b:2 * b + 1],
                 scale_cols[:, 2 * b + 1:2 * b + 2]], axis=0)
            yv[slot, :, :] = xv[slot] * col * g
            store(b, slot).start()
            if b + 2 < nbb:
                load(b + 2, slot).start()
        store(nbb - 2, (nbb - 2) % 2).wait()
        store(nbb - 1, (nbb - 1) % 2).wait()

    return pl.pallas_call(
        body,
        out_shape=jax.ShapeDtypeStruct((m, n_local), jnp.float32),
        in_specs=[pl.BlockSpec(memory_space=pl.ANY),
                  pl.BlockSpec(memory_space=pltpu.VMEM),
                  pl.BlockSpec(memory_space=pltpu.VMEM)],
        out_specs=pl.BlockSpec(memory_space=pl.ANY),
        scratch_shapes=[
            pltpu.VMEM((2, bmb, n_local), jnp.float32),
            pltpu.VMEM((2, bmb, n_local), jnp.float32),
            pltpu.SemaphoreType.DMA((2,)),
            pltpu.SemaphoreType.DMA((2,)),
        ],
        compiler_params=pltpu.CompilerParams(
            vmem_limit_bytes=100 * 1024 * 1024),
    )(x, total, g2)


def kernel(x, gamma):
    m, n_local = x.shape
    total = _partials_allreduce(x)
    return _normalize(x, total, gamma.reshape(1, n_local))
